# baseline (speedup 1.0000x reference)
import functools

import jax
import jax.numpy as jnp
from jax import lax
from jax.experimental import pallas as pl
from jax.experimental.pallas import tpu as pltpu
from jax.experimental.pallas import tpu_sc as plsc

NC = 2
NS = 16
L = 16
NW = NC * NS

D = 128
ROWS = 4096 * 200
ROWS_PER_W = ROWS // NW
CHUNK = 128
NCHUNK = ROWS_PER_W // CHUNK


def _make_sc_permute():
  mesh = plsc.VectorSubcoreMesh(core_axis_name="c", subcore_axis_name="s")

  @functools.partial(
      pl.kernel,
      mesh=mesh,
      out_type=jax.ShapeDtypeStruct((ROWS * D,), jnp.float32),
      scratch_types=[
          pltpu.VMEM((CHUNK * D,), jnp.float32),
          pltpu.VMEM((CHUNK * D,), jnp.float32),
          pltpu.VMEM((D,), jnp.int32),
      ],
      compiler_params=pltpu.CompilerParams(needs_layout_passes=False),
  )
  def permute_kernel(x_hbm, perm_hbm, out_hbm, inb, outb, permb):
    wid = lax.axis_index("s") * NC + lax.axis_index("c")
    base = wid * ROWS_PER_W * D

    pltpu.sync_copy(perm_hbm, permb)
    perm_vecs = [permb[pl.ds(c * L, L)] for c in range(D // L)]

    def chunk_body(g, carry):
      off = base + g * (CHUNK * D)
      pltpu.sync_copy(x_hbm.at[pl.ds(off, CHUNK * D)], inb)

      def row_body(r, carry2):
        rb = r * D
        for c in range(D // L):
          vals = plsc.load_gather(inb, [perm_vecs[c] + rb])
          outb[pl.ds(rb + c * L, L)] = vals
        return carry2

      lax.fori_loop(0, CHUNK, row_body, 0, unroll=2)
      pltpu.sync_copy(outb, out_hbm.at[pl.ds(off, CHUNK * D)])
      return carry

    lax.fori_loop(0, NCHUNK, chunk_body, 0)

  return permute_kernel


_sc_permute = _make_sc_permute()


def kernel(input, permutation):
  x_flat = input.reshape(ROWS * D)
  out_flat = _sc_permute(x_flat, permutation)
  return out_flat.reshape(input.shape)


# speedup vs baseline: 1.5105x; 1.5105x over previous
import functools

import jax
import jax.numpy as jnp
from jax import lax
from jax.experimental import pallas as pl
from jax.experimental.pallas import tpu as pltpu
from jax.experimental.pallas import tpu_sc as plsc

NC = 2
NS = 16
L = 16
NW = NC * NS

D = 128
ROWS = 4096 * 200
ROWS_PER_W = ROWS // NW
CHUNK = 128
NCHUNK = ROWS_PER_W // CHUNK
CB = CHUNK * D


def _make_sc_permute():
  mesh = plsc.VectorSubcoreMesh(core_axis_name="c", subcore_axis_name="s")

  @functools.partial(
      pl.kernel,
      mesh=mesh,
      out_type=jax.ShapeDtypeStruct((ROWS * D,), jnp.float32),
      scratch_types=[
          pltpu.VMEM((CB,), jnp.float32),
          pltpu.VMEM((CB,), jnp.float32),
          pltpu.VMEM((CB,), jnp.float32),
          pltpu.VMEM((CB,), jnp.float32),
          pltpu.VMEM((D,), jnp.int32),
          pltpu.SemaphoreType.DMA,
          pltpu.SemaphoreType.DMA,
          pltpu.SemaphoreType.DMA,
          pltpu.SemaphoreType.DMA,
      ],
      compiler_params=pltpu.CompilerParams(needs_layout_passes=False),
  )
  def permute_kernel(x_hbm, perm_hbm, out_hbm, ib0, ib1, ob0, ob1,
                     permb, si0, si1, so0, so1):
    wid = lax.axis_index("s") * NC + lax.axis_index("c")
    base = wid * ROWS_PER_W * D

    pltpu.sync_copy(perm_hbm, permb)
    perm_vecs = [permb[pl.ds(c * L, L)] for c in range(D // L)]

    def permute_chunk(ib, ob):
      def row_body(r, carry2):
        rb = r * D
        for c in range(D // L):
          ob[pl.ds(rb + c * L, L)] = plsc.load_gather(ib, [perm_vecs[c] + rb])
        return carry2

      lax.fori_loop(0, CHUNK, row_body, 0, unroll=2)

    def start_in(g, ib, sem):
      pltpu.async_copy(x_hbm.at[pl.ds(base + g * CB, CB)], ib, sem)

    def start_out(g, ob, sem):
      pltpu.async_copy(ob, out_hbm.at[pl.ds(base + g * CB, CB)], sem)

    def wait_in(ib, sem):
      pltpu.make_async_copy(x_hbm.at[pl.ds(base, CB)], ib, sem).wait()

    def wait_out(ob, sem):
      pltpu.make_async_copy(ob, out_hbm.at[pl.ds(base, CB)], sem).wait()

    start_in(0, ib0, si0)
    start_in(1, ib1, si1)

    def pair_body(i, carry):
      g = i * 2

      @pl.when(i > 0)
      def _():
        wait_out(ob0, so0)

      wait_in(ib0, si0)
      permute_chunk(ib0, ob0)
      start_out(g, ob0, so0)

      @pl.when(g + 2 < NCHUNK)
      def _():
        start_in(g + 2, ib0, si0)

      @pl.when(i > 0)
      def _():
        wait_out(ob1, so1)

      wait_in(ib1, si1)
      permute_chunk(ib1, ob1)
      start_out(g + 1, ob1, so1)

      @pl.when(g + 3 < NCHUNK)
      def _():
        start_in(g + 3, ib1, si1)

      return carry

    lax.fori_loop(0, NCHUNK // 2, pair_body, 0)
    wait_out(ob0, so0)
    wait_out(ob1, so1)

  return permute_kernel


_sc_permute = _make_sc_permute()


def kernel(input, permutation):
  x_flat = input.reshape(ROWS * D)
  out_flat = _sc_permute(x_flat, permutation)
  return out_flat.reshape(input.shape)


# speedup vs baseline: 4.0308x; 2.6685x over previous
import functools

import jax
import jax.numpy as jnp
from jax import lax
from jax.experimental import pallas as pl
from jax.experimental.pallas import tpu as pltpu
from jax.experimental.pallas import tpu_sc as plsc

NC = 2
NS = 16
L = 16
NW = NC * NS

D = 128
ROWS = 4096 * 200
ROWS_PER_W = ROWS // NW
CHUNK = 128
NCHUNK = ROWS_PER_W // CHUNK
CB = CHUNK * D


def _make_sc_permute():
  mesh = plsc.VectorSubcoreMesh(core_axis_name="c", subcore_axis_name="s")

  @functools.partial(
      pl.kernel,
      mesh=mesh,
      out_type=jax.ShapeDtypeStruct((ROWS * D,), jnp.float32),
      scratch_types=[
          pltpu.VMEM((CB,), jnp.float32),
          pltpu.VMEM((CB,), jnp.float32),
          pltpu.VMEM((CB,), jnp.float32),
          pltpu.VMEM((CB,), jnp.float32),
          pltpu.VMEM((D,), jnp.int32),
          pltpu.SemaphoreType.DMA,
          pltpu.SemaphoreType.DMA,
          pltpu.SemaphoreType.DMA,
          pltpu.SemaphoreType.DMA,
      ],
      compiler_params=pltpu.CompilerParams(needs_layout_passes=False),
  )
  def permute_kernel(x_hbm, perm_hbm, out_hbm, ib0, ib1, ob0, ob1,
                     permb, si0, si1, so0, so1):
    wid = lax.axis_index("s") * NC + lax.axis_index("c")
    base = wid * ROWS_PER_W * D

    pltpu.sync_copy(perm_hbm, permb)
    perm_vecs = [permb[pl.ds(c * L, L)] for c in range(D // L)]

    def permute_chunk(ib, ob):
      @plsc.parallel_loop(0, CHUNK, unroll=4)
      def row_body(r):
        rb = r * D
        for c in range(D // L):
          ob[pl.ds(rb + c * L, L)] = plsc.load_gather(ib, [perm_vecs[c] + rb])

    def start_in(g, ib, sem):
      pltpu.async_copy(x_hbm.at[pl.ds(base + g * CB, CB)], ib, sem)

    def start_out(g, ob, sem):
      pltpu.async_copy(ob, out_hbm.at[pl.ds(base + g * CB, CB)], sem)

    def wait_in(ib, sem):
      pltpu.make_async_copy(x_hbm.at[pl.ds(base, CB)], ib, sem).wait()

    def wait_out(ob, sem):
      pltpu.make_async_copy(ob, out_hbm.at[pl.ds(base, CB)], sem).wait()

    start_in(0, ib0, si0)
    start_in(1, ib1, si1)

    def pair_body(i, carry):
      g = i * 2

      @pl.when(i > 0)
      def _():
        wait_out(ob0, so0)

      wait_in(ib0, si0)
      permute_chunk(ib0, ob0)
      start_out(g, ob0, so0)

      @pl.when(g + 2 < NCHUNK)
      def _():
        start_in(g + 2, ib0, si0)

      @pl.when(i > 0)
      def _():
        wait_out(ob1, so1)

      wait_in(ib1, si1)
      permute_chunk(ib1, ob1)
      start_out(g + 1, ob1, so1)

      @pl.when(g + 3 < NCHUNK)
      def _():
        start_in(g + 3, ib1, si1)

      return carry

    lax.fori_loop(0, NCHUNK // 2, pair_body, 0)
    wait_out(ob0, so0)
    wait_out(ob1, so1)

  return permute_kernel


_sc_permute = _make_sc_permute()


def kernel(input, permutation):
  x_flat = input.reshape(ROWS * D)
  out_flat = _sc_permute(x_flat, permutation)
  return out_flat.reshape(input.shape)
